# baseline (speedup 1.0000x reference)
import jax
import jax.numpy as jnp
from jax.experimental import pallas as pl
from jax.experimental.pallas import tpu as pltpu


def _dma_copy_body(x_ref, b_ref, xo_ref, bo_ref, sem_x, sem_b):
    cx = pltpu.make_async_copy(x_ref, xo_ref, sem_x)
    cb = pltpu.make_async_copy(b_ref, bo_ref, sem_b)
    cx.start()
    cb.start()
    cx.wait()
    cb.wait()


def kernel(x, edge_index, batch):
    del edge_index
    hbm = pl.BlockSpec(memory_space=pltpu.MemorySpace.HBM)
    xo, bo = pl.pallas_call(
        _dma_copy_body,
        in_specs=[hbm, hbm],
        out_specs=(hbm, hbm),
        out_shape=(
            jax.ShapeDtypeStruct(x.shape, x.dtype),
            jax.ShapeDtypeStruct(batch.shape, batch.dtype),
        ),
        scratch_shapes=[pltpu.SemaphoreType.DMA, pltpu.SemaphoreType.DMA],
    )(x, batch)
    return (xo, bo)


# speedup vs baseline: 12.9141x; 12.9141x over previous
import jax
import jax.numpy as jnp
from jax.experimental import pallas as pl

_GRID = 10


def _copy_body(x_ref, b_ref, xo_ref, bo_ref):
    xo_ref[...] = x_ref[...]
    bo_ref[...] = b_ref[...]


def kernel(x, edge_index, batch):
    del edge_index
    n, d = x.shape
    rows = n // _GRID
    b3 = batch.reshape(_GRID, 1, rows)
    xo, bo = pl.pallas_call(
        _copy_body,
        grid=(_GRID,),
        in_specs=[
            pl.BlockSpec((rows, d), lambda i: (i, 0)),
            pl.BlockSpec((1, 1, rows), lambda i: (i, 0, 0)),
        ],
        out_specs=(
            pl.BlockSpec((rows, d), lambda i: (i, 0)),
            pl.BlockSpec((1, 1, rows), lambda i: (i, 0, 0)),
        ),
        out_shape=(
            jax.ShapeDtypeStruct(x.shape, x.dtype),
            jax.ShapeDtypeStruct(b3.shape, b3.dtype),
        ),
    )(x, b3)
    return (xo, bo.reshape(batch.shape))


# speedup vs baseline: 22.1840x; 1.7178x over previous
import jax
import jax.numpy as jnp
from jax.experimental import pallas as pl


def _copy_body(x_ref, xo_ref):
    xo_ref[...] = x_ref[...]


def _copy_body_b(b_ref, bo_ref):
    bo_ref[...] = b_ref[...]


def kernel(x, edge_index, batch):
    del edge_index
    xo = pl.pallas_call(
        _copy_body,
        out_shape=jax.ShapeDtypeStruct(x.shape, x.dtype),
    )(x)
    bo = pl.pallas_call(
        _copy_body_b,
        out_shape=jax.ShapeDtypeStruct(batch.shape, batch.dtype),
    )(batch)
    return (xo, bo)


# speedup vs baseline: 27.5115x; 1.2401x over previous
import jax
import jax.numpy as jnp
from jax.experimental import pallas as pl


def _copy_body(x_ref, b_ref, xo_ref, bo_ref):
    xo_ref[...] = x_ref[...]
    bo_ref[...] = b_ref[...]


def kernel(x, edge_index, batch):
    del edge_index
    xo, bo = pl.pallas_call(
        _copy_body,
        out_shape=(
            jax.ShapeDtypeStruct(x.shape, x.dtype),
            jax.ShapeDtypeStruct(batch.shape, batch.dtype),
        ),
    )(x, batch)
    return (xo, bo)


# speedup vs baseline: 32.0807x; 1.1661x over previous
import jax
import jax.numpy as jnp
from jax.experimental import pallas as pl

_GRID = 2


def _copy_body(x_ref, b_ref, xo_ref, bo_ref):
    xo_ref[...] = x_ref[...]
    bo_ref[...] = b_ref[...]


def kernel(x, edge_index, batch):
    del edge_index
    n, d = x.shape
    rows = n // _GRID
    xo, bo = pl.pallas_call(
        _copy_body,
        grid=(_GRID,),
        in_specs=[
            pl.BlockSpec((rows, d), lambda i: (i, 0)),
            pl.BlockSpec(batch.shape, lambda i: (0,)),
        ],
        out_specs=(
            pl.BlockSpec((rows, d), lambda i: (i, 0)),
            pl.BlockSpec(batch.shape, lambda i: (0,)),
        ),
        out_shape=(
            jax.ShapeDtypeStruct(x.shape, x.dtype),
            jax.ShapeDtypeStruct(batch.shape, batch.dtype),
        ),
    )(x, batch)
    return (xo, bo)
